# baseline (speedup 1.0000x reference)
import functools

import jax
import jax.numpy as jnp
from jax import lax
from jax.experimental import pallas as pl
from jax.experimental.pallas import tpu as pltpu
from jax.experimental.pallas import tpu_sc as plsc

_B, _S, _HID = 2, 2048, 2048
_H, _D, _R, _QLR, _TOPK = 16, 128, 64, 1536, 2048
_T = _S


def _q_body(qr_ref, wqb_ref, m1_ref, m2_ref, out_ref):
    qt = lax.dot_general(wqb_ref[...], qr_ref[0], (((1,), (1,)), ((), ())),
                         preferred_element_type=jnp.float32)
    q = qt.T
    sq = q.shape[0]
    q = q.reshape(sq, _H, _D)
    qs = jnp.concatenate([q[:, :, 32:64], q[:, :, :32], q[:, :, 64:]], axis=-1)
    m1 = m1_ref[0].reshape(sq, 1, _D)
    m2 = m2_ref[0].reshape(sq, 1, _D)
    out_ref[0] = (q * m1 + qs * m2).reshape(sq, _H * _D)


def _k_body(h_ref, wk_ref, ww_ref, g_ref, b_ref, m1t_ref, m2t_ref,
            k_out, hw_out):
    h = h_ref[0]
    kt = lax.dot_general(wk_ref[...], h, (((1,), (1,)), ((), ())),
                         preferred_element_type=jnp.float32)
    mu = jnp.mean(kt, axis=0, keepdims=True)
    var = jnp.mean((kt - mu) ** 2, axis=0, keepdims=True)
    kt = (kt - mu) / jnp.sqrt(var + 1e-5) * g_ref[...].T + b_ref[...].T
    ks = jnp.concatenate([kt[32:64, :], kt[:32, :], kt[64:, :]], axis=0)
    k_out[0] = (kt * m1t_ref[0] + ks * m2t_ref[0]).T
    hw = lax.dot_general(ww_ref[...], h, (((1,), (1,)), ((), ())),
                         preferred_element_type=jnp.float32).T
    hw_out[0] = hw * (_H ** -0.5)


def _score_body(q_ref, k_ref, hw_ref, out_ref):
    q = q_ref[0]
    k = k_ref[0]
    hw = hw_ref[0]
    sq, tt = q.shape[0], k.shape[0]
    acc = jnp.zeros((sq, tt), jnp.float32)
    for h in range(_H):
        s = lax.dot_general(q[:, h * _D:(h + 1) * _D], k,
                            (((1,), (1,)), ((), ())),
                            preferred_element_type=jnp.float32)
        acc = acc + jnp.maximum(s * (hw[:, h][:, None] * (_D ** -0.5)), 0.0)
    out_ref[0] = acc


def _sort_body(s_ref, idx_ref):
    keys = s_ref[...]
    rt, n = keys.shape
    j = lax.broadcasted_iota(jnp.int32, (rt, n), 1)
    idx = j

    def cmpex(keys, idx, size, d):
        upper = (j & d) != 0
        desc = (j & size) != 0
        pk = jnp.where(upper, pltpu.roll(keys, d, 1), pltpu.roll(keys, -d, 1))
        pi = jnp.where(upper, pltpu.roll(idx, d, 1), pltpu.roll(idx, -d, 1))
        before = (keys > pk) | ((keys == pk) & (idx < pi))
        keep = before ^ upper ^ desc
        return jnp.where(keep, keys, pk), jnp.where(keep, idx, pi)

    def outer(k, carry):
        keys, idx = carry
        size = jnp.int32(1) << k

        def inner(st, carry):
            keys, idx = carry
            d = jnp.int32(1) << (k - 1 - st)
            return cmpex(keys, idx, size, d)

        return lax.fori_loop(0, k, inner, (keys, idx))

    keys, idx = lax.fori_loop(1, 12, outer, (keys, idx))
    idx_ref[...] = idx


def _build_rope_mults(cos, sin):
    c = cos[:, :, 0, :]
    s = sin[:, :, 0, :]
    ones = jnp.ones((_B, _S, _D - _R), jnp.float32)
    zeros = jnp.zeros((_B, _S, _D - _R), jnp.float32)
    m1 = jnp.concatenate([c, c, ones], axis=-1)
    m2 = jnp.concatenate([-s, s, zeros], axis=-1)
    return m1, m2


def _q_stage(q_resid, W_qb, m1, m2):
    sq_a = 512
    return pl.pallas_call(
        _q_body,
        grid=(_B, _S // sq_a),
        in_specs=[
            pl.BlockSpec((1, sq_a, _QLR), lambda b, i: (b, i, 0)),
            pl.BlockSpec((_H * _D, _QLR), lambda b, i: (0, 0)),
            pl.BlockSpec((1, sq_a, _D), lambda b, i: (b, i, 0)),
            pl.BlockSpec((1, sq_a, _D), lambda b, i: (b, i, 0)),
        ],
        out_specs=pl.BlockSpec((1, sq_a, _H * _D), lambda b, i: (b, i, 0)),
        out_shape=jax.ShapeDtypeStruct((_B, _S, _H * _D), jnp.float32),
    )(q_resid, W_qb, m1, m2)


def _k_stage(hidden_states, W_k, W_w, ln_g, ln_b, m1, m2):
    sk = 512
    m1t = m1.transpose(0, 2, 1)
    m2t = m2.transpose(0, 2, 1)
    return pl.pallas_call(
        _k_body,
        grid=(_B, _S // sk),
        in_specs=[
            pl.BlockSpec((1, sk, _HID), lambda b, i: (b, i, 0)),
            pl.BlockSpec((_D, _HID), lambda b, i: (0, 0)),
            pl.BlockSpec((_H, _HID), lambda b, i: (0, 0)),
            pl.BlockSpec((1, _D), lambda b, i: (0, 0)),
            pl.BlockSpec((1, _D), lambda b, i: (0, 0)),
            pl.BlockSpec((1, _D, sk), lambda b, i: (b, 0, i)),
            pl.BlockSpec((1, _D, sk), lambda b, i: (b, 0, i)),
        ],
        out_specs=[
            pl.BlockSpec((1, sk, _D), lambda b, i: (b, i, 0)),
            pl.BlockSpec((1, sk, _H), lambda b, i: (b, i, 0)),
        ],
        out_shape=[
            jax.ShapeDtypeStruct((_B, _S, _D), jnp.float32),
            jax.ShapeDtypeStruct((_B, _S, _H), jnp.float32),
        ],
    )(hidden_states, W_k, W_w, ln_g.reshape(1, _D), ln_b.reshape(1, _D),
      m1t, m2t)


def _score_stage(q_roped, k_roped, head_w):
    sq_c, tt = 256, 512
    return pl.pallas_call(
        _score_body,
        grid=(_B, _S // sq_c, _T // tt),
        in_specs=[
            pl.BlockSpec((1, sq_c, _H * _D), lambda b, i, t: (b, i, 0)),
            pl.BlockSpec((1, tt, _D), lambda b, i, t: (b, t, 0)),
            pl.BlockSpec((1, sq_c, _H), lambda b, i, t: (b, i, 0)),
        ],
        out_specs=pl.BlockSpec((1, sq_c, tt), lambda b, i, t: (b, i, t)),
        out_shape=jax.ShapeDtypeStruct((_B, _S, _T), jnp.float32),
    )(q_roped, k_roped, head_w)


def _sort_stage(scores):
    rows = _B * _S
    rt = 256
    idx = pl.pallas_call(
        _sort_body,
        grid=(rows // rt,),
        in_specs=[pl.BlockSpec((rt, _T), lambda i: (i, 0))],
        out_specs=pl.BlockSpec((rt, _T), lambda i: (i, 0)),
        out_shape=jax.ShapeDtypeStruct((rows, _T), jnp.int32),
    )(scores.reshape(rows, _T))

    return idx.reshape(_B, _S, _T)


def _score_body_t(q_ref, k_ref, hw_ref, out_ref):
    q = q_ref[0]
    k = k_ref[0]
    hw = hw_ref[0]
    sq, tt = q.shape[0], k.shape[0]
    acc = jnp.zeros((sq, tt), jnp.float32)
    for h in range(_H):
        s = lax.dot_general(q[:, h * _D:(h + 1) * _D], k,
                            (((1,), (1,)), ((), ())),
                            preferred_element_type=jnp.float32)
        acc = acc + jnp.maximum(s * (hw[:, h][:, None] * (_D ** -0.5)), 0.0)
    out_ref[...] = acc.T


def _score_stage_t(q_roped, k_roped, head_w):
    sq_c, tt = 256, 512
    nsb = _S // sq_c
    return pl.pallas_call(
        _score_body_t,
        grid=(_B, nsb, _T // tt),
        in_specs=[
            pl.BlockSpec((1, sq_c, _H * _D), lambda b, i, t: (b, i, 0)),
            pl.BlockSpec((1, tt, _D), lambda b, i, t: (b, t, 0)),
            pl.BlockSpec((1, sq_c, _H), lambda b, i, t: (b, i, 0)),
        ],
        out_specs=pl.BlockSpec((tt, sq_c),
                               lambda b, i, t: (t, b * (_S // 256) + i)),
        out_shape=jax.ShapeDtypeStruct((_T, _B * _S), jnp.float32),
    )(q_roped, k_roped, head_w)


def _sc_sort_stage(scores_t):
    rows = _B * _S
    nc, ns = 2, 16
    nw = nc * ns
    rpw = rows // nw
    groups = rpw // 16
    chs = 128
    nch = _T // chs
    mesh = plsc.VectorSubcoreMesh(core_axis_name="c", subcore_axis_name="s")

    @functools.partial(
        pl.kernel,
        out_type=jax.ShapeDtypeStruct((nw, rpw * _T), jnp.int32),
        mesh=mesh,
        compiler_params=pltpu.CompilerParams(
            use_tc_tiling_on_sc=False, needs_layout_passes=False),
        scratch_types=[
            pltpu.VMEM((chs, rpw), jnp.float32),
            pltpu.VMEM((_T * 16,), jnp.float32),
            pltpu.VMEM((_T * 16,), jnp.int32),
            pltpu.VMEM((_T * 16,), jnp.int32),
            pltpu.VMEM((128 * 16,), jnp.int32),
        ],
    )
    def sortk(scores_hbm, out_hbm, staging, keyb, vala, valb, hist):
        wid = lax.axis_index("s") * nc + lax.axis_index("c")
        lane = lax.broadcasted_iota(jnp.int32, (16,), 0)
        zeros16 = jnp.zeros((16,), jnp.int32)
        ones16 = jnp.ones((16,), jnp.int32)

        def radix_pass(shift, vin, vout, out_rowmajor=False):
            def zh(d, c):
                keyb_addr = jnp.full((16,), d * 16, jnp.int32) + lane
                plsc.store_scatter(hist, [keyb_addr], zeros16)
                return c
            lax.fori_loop(0, 128, zh, 0, unroll=8)

            def digit_of(p):
                if vin is None:
                    v = jnp.full((16,), p, jnp.int32)
                else:
                    v = plsc.load_gather(
                        vin, [jnp.full((16,), p * 16, jnp.int32) + lane])
                kf = plsc.load_gather(keyb, [v * 16 + lane])
                kb = plsc.bitcast(kf, jnp.int32)
                d = ((kb >> shift) & 127) ^ 127
                return v, d

            def cnt(p, c):
                _, d = digit_of(p)
                plsc.addupdate_scatter(hist, [d * 16 + lane], ones16)
                return c
            lax.fori_loop(0, _T, cnt, 0, unroll=4)

            def sc_scan(d, run):
                dv = jnp.full((16,), d * 16, jnp.int32) + lane
                c = plsc.load_gather(hist, [dv])
                plsc.store_scatter(hist, [dv], run)
                return run + c
            lax.fori_loop(0, 128, sc_scan, zeros16, unroll=4)

            def perm(p, c):
                v, d = digit_of(p)
                da = d * 16 + lane
                pos = plsc.load_gather(hist, [da])
                plsc.addupdate_scatter(hist, [da], ones16)
                if out_rowmajor:
                    plsc.store_scatter(vout, [lane * _T + pos], v)
                else:
                    plsc.store_scatter(vout, [pos * 16 + lane], v)
                return c
            lax.fori_loop(0, _T, perm, 0, unroll=4)

        def one_group(g, c):

            def load_chunk(ch, cc):
                pltpu.sync_copy(
                    scores_hbm.at[pl.ds(ch * chs, chs),
                                  pl.ds(wid * rpw, rpw)], staging)

                def xf(p, c2):
                    v = staging[p, pl.ds(g * 16, 16)]
                    keyb[pl.ds((ch * chs + p) * 16, 16)] = v
                    return c2
                lax.fori_loop(0, chs, xf, 0, unroll=8)
                return cc
            lax.fori_loop(0, nch, load_chunk, 0)

            radix_pass(0, None, vala)
            radix_pass(7, vala, valb)
            radix_pass(14, valb, vala)
            radix_pass(21, vala, valb)
            radix_pass(28, valb, vala, out_rowmajor=True)

            pltpu.sync_copy(vala, out_hbm.at[wid, pl.ds(g * 16 * _T, 16 * _T)])
            return c

        lax.fori_loop(0, groups, one_group, 0)

    return sortk(scores_t)


def _unused_tr():
    pass


def _stages_for_probe(hidden_states, q_resid, cos, sin, W_qb, W_k,
                      ln_g, ln_b, W_w):
    m1, m2 = _build_rope_mults(cos, sin)
    q_roped = _q_stage(q_resid, W_qb, m1, m2)
    k_roped, head_w = _k_stage(hidden_states, W_k, W_w, ln_g, ln_b, m1, m2)
    scores = _score_stage(q_roped, k_roped, head_w)
    return q_roped, k_roped, head_w, scores


def kernel(hidden_states, q_resid, cos, sin, attention_mask, cache_position,
           W_qb, W_k, ln_g, ln_b, W_w):
    del attention_mask, cache_position
    m1, m2 = _build_rope_mults(cos, sin)
    q_roped = _q_stage(q_resid, W_qb, m1, m2)
    k_roped, head_w = _k_stage(hidden_states, W_k, W_w, ln_g, ln_b, m1, m2)
    scores_t = _score_stage_t(q_roped, k_roped, head_w)
    idx = _sc_sort_stage(scores_t)
    return idx.reshape(_B, _S, _T)


# speedup vs baseline: 1.6386x; 1.6386x over previous
import functools

import jax
import jax.numpy as jnp
from jax import lax
from jax.experimental import pallas as pl
from jax.experimental.pallas import tpu as pltpu
from jax.experimental.pallas import tpu_sc as plsc

_B, _S, _HID = 2, 2048, 2048
_H, _D, _R, _QLR, _TOPK = 16, 128, 64, 1536, 2048
_T = _S


def _q_body(qr_ref, wqb_ref, m1_ref, m2_ref, out_ref):
    qt = lax.dot_general(wqb_ref[...], qr_ref[0], (((1,), (1,)), ((), ())),
                         preferred_element_type=jnp.float32)
    q = qt.T
    sq = q.shape[0]
    q = q.reshape(sq, _H, _D)
    qs = jnp.concatenate([q[:, :, 32:64], q[:, :, :32], q[:, :, 64:]], axis=-1)
    m1 = m1_ref[0].reshape(sq, 1, _D)
    m2 = m2_ref[0].reshape(sq, 1, _D)
    out_ref[0] = (q * m1 + qs * m2).reshape(sq, _H * _D)


def _k_body(h_ref, wk_ref, ww_ref, g_ref, b_ref, m1t_ref, m2t_ref,
            k_out, hw_out):
    h = h_ref[0]
    kt = lax.dot_general(wk_ref[...], h, (((1,), (1,)), ((), ())),
                         preferred_element_type=jnp.float32)
    mu = jnp.mean(kt, axis=0, keepdims=True)
    var = jnp.mean((kt - mu) ** 2, axis=0, keepdims=True)
    kt = (kt - mu) / jnp.sqrt(var + 1e-5) * g_ref[...].T + b_ref[...].T
    ks = jnp.concatenate([kt[32:64, :], kt[:32, :], kt[64:, :]], axis=0)
    k_out[0] = (kt * m1t_ref[0] + ks * m2t_ref[0]).T
    hw = lax.dot_general(ww_ref[...], h, (((1,), (1,)), ((), ())),
                         preferred_element_type=jnp.float32).T
    hw_out[0] = hw * (_H ** -0.5)


def _score_body(q_ref, k_ref, hw_ref, out_ref):
    q = q_ref[0]
    k = k_ref[0]
    hw = hw_ref[0]
    sq, tt = q.shape[0], k.shape[0]
    acc = jnp.zeros((sq, tt), jnp.float32)
    for h in range(_H):
        s = lax.dot_general(q[:, h * _D:(h + 1) * _D], k,
                            (((1,), (1,)), ((), ())),
                            preferred_element_type=jnp.float32)
        acc = acc + jnp.maximum(s * (hw[:, h][:, None] * (_D ** -0.5)), 0.0)
    out_ref[0] = acc


def _sort_body(s_ref, idx_ref):
    keys = s_ref[...]
    rt, n = keys.shape
    j = lax.broadcasted_iota(jnp.int32, (rt, n), 1)
    idx = j

    def cmpex(keys, idx, size, d):
        upper = (j & d) != 0
        desc = (j & size) != 0
        pk = jnp.where(upper, pltpu.roll(keys, d, 1), pltpu.roll(keys, -d, 1))
        pi = jnp.where(upper, pltpu.roll(idx, d, 1), pltpu.roll(idx, -d, 1))
        before = (keys > pk) | ((keys == pk) & (idx < pi))
        keep = before ^ upper ^ desc
        return jnp.where(keep, keys, pk), jnp.where(keep, idx, pi)

    def outer(k, carry):
        keys, idx = carry
        size = jnp.int32(1) << k

        def inner(st, carry):
            keys, idx = carry
            d = jnp.int32(1) << (k - 1 - st)
            return cmpex(keys, idx, size, d)

        return lax.fori_loop(0, k, inner, (keys, idx))

    keys, idx = lax.fori_loop(1, 12, outer, (keys, idx))
    idx_ref[...] = idx


def _build_rope_mults(cos, sin):
    c = cos[:, :, 0, :]
    s = sin[:, :, 0, :]
    ones = jnp.ones((_B, _S, _D - _R), jnp.float32)
    zeros = jnp.zeros((_B, _S, _D - _R), jnp.float32)
    m1 = jnp.concatenate([c, c, ones], axis=-1)
    m2 = jnp.concatenate([-s, s, zeros], axis=-1)
    return m1, m2


def _q_stage(q_resid, W_qb, m1, m2):
    sq_a = 512
    return pl.pallas_call(
        _q_body,
        grid=(_B, _S // sq_a),
        in_specs=[
            pl.BlockSpec((1, sq_a, _QLR), lambda b, i: (b, i, 0)),
            pl.BlockSpec((_H * _D, _QLR), lambda b, i: (0, 0)),
            pl.BlockSpec((1, sq_a, _D), lambda b, i: (b, i, 0)),
            pl.BlockSpec((1, sq_a, _D), lambda b, i: (b, i, 0)),
        ],
        out_specs=pl.BlockSpec((1, sq_a, _H * _D), lambda b, i: (b, i, 0)),
        out_shape=jax.ShapeDtypeStruct((_B, _S, _H * _D), jnp.float32),
    )(q_resid, W_qb, m1, m2)


def _k_stage(hidden_states, W_k, W_w, ln_g, ln_b, m1, m2):
    sk = 512
    m1t = m1.transpose(0, 2, 1)
    m2t = m2.transpose(0, 2, 1)
    return pl.pallas_call(
        _k_body,
        grid=(_B, _S // sk),
        in_specs=[
            pl.BlockSpec((1, sk, _HID), lambda b, i: (b, i, 0)),
            pl.BlockSpec((_D, _HID), lambda b, i: (0, 0)),
            pl.BlockSpec((_H, _HID), lambda b, i: (0, 0)),
            pl.BlockSpec((1, _D), lambda b, i: (0, 0)),
            pl.BlockSpec((1, _D), lambda b, i: (0, 0)),
            pl.BlockSpec((1, _D, sk), lambda b, i: (b, 0, i)),
            pl.BlockSpec((1, _D, sk), lambda b, i: (b, 0, i)),
        ],
        out_specs=[
            pl.BlockSpec((1, sk, _D), lambda b, i: (b, i, 0)),
            pl.BlockSpec((1, sk, _H), lambda b, i: (b, i, 0)),
        ],
        out_shape=[
            jax.ShapeDtypeStruct((_B, _S, _D), jnp.float32),
            jax.ShapeDtypeStruct((_B, _S, _H), jnp.float32),
        ],
    )(hidden_states, W_k, W_w, ln_g.reshape(1, _D), ln_b.reshape(1, _D),
      m1t, m2t)


def _score_stage(q_roped, k_roped, head_w):
    sq_c, tt = 256, 512
    return pl.pallas_call(
        _score_body,
        grid=(_B, _S // sq_c, _T // tt),
        in_specs=[
            pl.BlockSpec((1, sq_c, _H * _D), lambda b, i, t: (b, i, 0)),
            pl.BlockSpec((1, tt, _D), lambda b, i, t: (b, t, 0)),
            pl.BlockSpec((1, sq_c, _H), lambda b, i, t: (b, i, 0)),
        ],
        out_specs=pl.BlockSpec((1, sq_c, tt), lambda b, i, t: (b, i, t)),
        out_shape=jax.ShapeDtypeStruct((_B, _S, _T), jnp.float32),
    )(q_roped, k_roped, head_w)


def _sort_stage(scores):
    rows = _B * _S
    rt = 256
    idx = pl.pallas_call(
        _sort_body,
        grid=(rows // rt,),
        in_specs=[pl.BlockSpec((rt, _T), lambda i: (i, 0))],
        out_specs=pl.BlockSpec((rt, _T), lambda i: (i, 0)),
        out_shape=jax.ShapeDtypeStruct((rows, _T), jnp.int32),
    )(scores.reshape(rows, _T))

    return idx.reshape(_B, _S, _T)


def _score_body_t(q_ref, k_ref, hw_ref, out_ref):
    q = q_ref[0]
    k = k_ref[0]
    hw = hw_ref[0]
    sq, tt = q.shape[0], k.shape[0]
    acc = jnp.zeros((sq, tt), jnp.float32)
    for h in range(_H):
        s = lax.dot_general(q[:, h * _D:(h + 1) * _D], k,
                            (((1,), (1,)), ((), ())),
                            preferred_element_type=jnp.float32)
        acc = acc + jnp.maximum(s * (hw[:, h][:, None] * (_D ** -0.5)), 0.0)
    out_ref[...] = acc.T


def _score_stage_t(q_roped, k_roped, head_w):
    sq_c, tt = 256, 512
    nsb = _S // sq_c
    return pl.pallas_call(
        _score_body_t,
        grid=(_B, nsb, _T // tt),
        in_specs=[
            pl.BlockSpec((1, sq_c, _H * _D), lambda b, i, t: (b, i, 0)),
            pl.BlockSpec((1, tt, _D), lambda b, i, t: (b, t, 0)),
            pl.BlockSpec((1, sq_c, _H), lambda b, i, t: (b, i, 0)),
        ],
        out_specs=pl.BlockSpec((tt, sq_c),
                               lambda b, i, t: (t, b * (_S // 256) + i)),
        out_shape=jax.ShapeDtypeStruct((_T, _B * _S), jnp.float32),
    )(q_roped, k_roped, head_w)


def _sc_sort_stage(scores_t):
    rows = _B * _S
    nc, ns = 2, 16
    nw = nc * ns
    rpw = rows // nw
    groups = rpw // 16
    chs = 64
    nch = _T // chs
    RB, RMASK = 9, 511
    mesh = plsc.VectorSubcoreMesh(core_axis_name="c", subcore_axis_name="s")

    @functools.partial(
        pl.kernel,
        out_type=jax.ShapeDtypeStruct((nw, rpw * _T), jnp.int32),
        mesh=mesh,
        compiler_params=pltpu.CompilerParams(
            use_tc_tiling_on_sc=False, needs_layout_passes=False),
        scratch_types=[
            pltpu.VMEM((chs, rpw), jnp.float32),
            pltpu.VMEM((_T * 16,), jnp.float32),
            pltpu.VMEM((_T * 16,), jnp.int32),
            pltpu.VMEM((_T * 16,), jnp.int32),
            pltpu.VMEM((512 * 16,), jnp.int32),
            pltpu.VMEM((512 * 16,), jnp.int32),
        ],
    )
    def sortk(scores_hbm, out_hbm, staging, keyb, vala, valb, ha, hb):
        wid = lax.axis_index("s") * nc + lax.axis_index("c")
        lane = lax.broadcasted_iota(jnp.int32, (16,), 0)
        zeros16 = jnp.zeros((16,), jnp.int32)
        ones16 = jnp.ones((16,), jnp.int32)

        def zero_hist(h):
            def zh(d, c):
                h[pl.ds(d * 16, 16)] = zeros16
                return c
            lax.fori_loop(0, 512, zh, 0, unroll=8)

        def scan_hist(h):
            def sc(d, run):
                c = h[pl.ds(d * 16, 16)]
                h[pl.ds(d * 16, 16)] = run
                return run + c
            lax.fori_loop(0, 512, sc, zeros16, unroll=8)

        def count0(h):
            def cnt(p, c):
                kb = plsc.bitcast(keyb[pl.ds(p * 16, 16)], jnp.int32)
                d = (kb & RMASK) ^ RMASK
                plsc.addupdate_scatter(h, [d * 16 + lane], ones16)
                return c
            lax.fori_loop(0, _T, cnt, 0, unroll=8)

        def perm(shift, vin, vout, hcur, nshift, hnxt, rowmajor):
            def body(p, c):
                if vin is None:
                    v = jnp.full((16,), p, jnp.int32)
                    kb = plsc.bitcast(keyb[pl.ds(p * 16, 16)], jnp.int32)
                else:
                    v = plsc.load_gather(
                        vin, [jnp.full((16,), p * 16, jnp.int32) + lane])
                    kb = plsc.bitcast(
                        plsc.load_gather(keyb, [v * 16 + lane]), jnp.int32)
                d = ((kb >> shift) & RMASK) ^ RMASK
                da = d * 16 + lane
                pos = plsc.load_gather(hcur, [da])
                plsc.addupdate_scatter(hcur, [da], ones16)
                if rowmajor:
                    plsc.store_scatter(vout, [lane * _T + pos], v)
                else:
                    plsc.store_scatter(vout, [pos * 16 + lane], v)
                if hnxt is not None:
                    d2 = ((kb >> nshift) & RMASK) ^ RMASK
                    plsc.addupdate_scatter(hnxt, [d2 * 16 + lane], ones16)
                return c
            lax.fori_loop(0, _T, body, 0, unroll=8)

        def one_group(g, c):
            def load_chunk(ch, cc):
                pltpu.sync_copy(
                    scores_hbm.at[pl.ds(ch * chs, chs),
                                  pl.ds(wid * rpw, rpw)], staging)

                def xf(p, c2):
                    keyb[pl.ds((ch * chs + p) * 16, 16)] = (
                        staging[p, pl.ds(g * 16, 16)])
                    return c2
                lax.fori_loop(0, chs, xf, 0, unroll=8)
                return cc
            lax.fori_loop(0, nch, load_chunk, 0)

            zero_hist(ha)
            count0(ha)
            scan_hist(ha)
            zero_hist(hb)
            perm(0, None, vala, ha, RB, hb, False)
            scan_hist(hb)
            zero_hist(ha)
            perm(RB, vala, valb, hb, 2 * RB, ha, False)
            scan_hist(ha)
            zero_hist(hb)
            perm(2 * RB, valb, vala, ha, 3 * RB, hb, False)
            scan_hist(hb)
            perm(3 * RB, vala, valb, hb, 0, None, True)

            pltpu.sync_copy(valb, out_hbm.at[wid, pl.ds(g * 16 * _T, 16 * _T)])
            return c

        lax.fori_loop(0, groups, one_group, 0)

    return sortk(scores_t)


def _unused_tr():
    pass


def _stages_for_probe(hidden_states, q_resid, cos, sin, W_qb, W_k,
                      ln_g, ln_b, W_w):
    m1, m2 = _build_rope_mults(cos, sin)
    q_roped = _q_stage(q_resid, W_qb, m1, m2)
    k_roped, head_w = _k_stage(hidden_states, W_k, W_w, ln_g, ln_b, m1, m2)
    scores = _score_stage(q_roped, k_roped, head_w)
    return q_roped, k_roped, head_w, scores


def kernel(hidden_states, q_resid, cos, sin, attention_mask, cache_position,
           W_qb, W_k, ln_g, ln_b, W_w):
    del attention_mask, cache_position
    m1, m2 = _build_rope_mults(cos, sin)
    q_roped = _q_stage(q_resid, W_qb, m1, m2)
    k_roped, head_w = _k_stage(hidden_states, W_k, W_w, ln_g, ln_b, m1, m2)
    scores_t = _score_stage_t(q_roped, k_roped, head_w)
    idx = _sc_sort_stage(scores_t)
    return idx.reshape(_B, _S, _T)
